# baseline (device time: 87520 ns/iter reference)
import jax
import jax.numpy as jnp
from jax import lax
from jax.experimental import pallas as pl
from jax.experimental.pallas import tpu as pltpu

N_DEV = 4


def kernel(x, w_mat):
    m_per, k = x.shape
    _, n_per = w_mat.shape

    def body(x_ref, w_ref, out_ref, comm_ref, w_bf_ref, send_sems, recv_sems):
        my = lax.axis_index("i")
        left = lax.rem(my + (N_DEV - 1), N_DEV)
        right = lax.rem(my + 1, N_DEV)

        barrier_sem = pltpu.get_barrier_semaphore()
        for nbr in (left, right):
            pl.semaphore_signal(
                barrier_sem, inc=1,
                device_id=(nbr,), device_id_type=pl.DeviceIdType.MESH,
            )
        pl.semaphore_wait(barrier_sem, 2)

        w_bf_ref[...] = w_ref[...].astype(jnp.bfloat16)
        comm_ref[my] = x_ref[...].astype(jnp.bfloat16)

        out_ref[pl.ds(my * m_per, m_per), :] = jnp.dot(
            comm_ref[my], w_bf_ref[...], preferred_element_type=jnp.float32
        )

        for h in range(N_DEV - 1):
            src = lax.rem(my - h + N_DEV, N_DEV)
            rdma = pltpu.make_async_remote_copy(
                src_ref=comm_ref.at[src],
                dst_ref=comm_ref.at[src],
                send_sem=send_sems.at[h],
                recv_sem=recv_sems.at[h],
                device_id=(right,),
                device_id_type=pl.DeviceIdType.MESH,
            )
            rdma.start()
            rdma.wait()
            origin = lax.rem(my - h - 1 + N_DEV, N_DEV)
            out_ref[pl.ds(origin * m_per, m_per), :] = jnp.dot(
                comm_ref[origin], w_bf_ref[...],
                preferred_element_type=jnp.float32,
            )

    return pl.pallas_call(
        body,
        out_shape=jax.ShapeDtypeStruct((N_DEV * m_per, n_per), jnp.float32),
        in_specs=[
            pl.BlockSpec(memory_space=pltpu.VMEM),
            pl.BlockSpec(memory_space=pltpu.VMEM),
        ],
        out_specs=pl.BlockSpec(memory_space=pltpu.VMEM),
        scratch_shapes=[
            pltpu.VMEM((N_DEV, m_per, k), jnp.bfloat16),
            pltpu.VMEM((k, n_per), jnp.bfloat16),
            pltpu.SemaphoreType.DMA((N_DEV - 1,)),
            pltpu.SemaphoreType.DMA((N_DEV - 1,)),
        ],
        compiler_params=pltpu.CompilerParams(collective_id=0),
    )(x, w_mat)


# device time: 47829 ns/iter; 1.8299x vs baseline; 1.8299x over previous
import jax
import jax.numpy as jnp
from jax import lax
from jax.experimental import pallas as pl
from jax.experimental.pallas import tpu as pltpu

N_DEV = 4


def kernel(x, w_mat):
    m_per, k = x.shape
    _, n_per = w_mat.shape
    hm = m_per // 2

    def body(x_ref, w_ref, out_ref, comm_ref, w_bf_ref, send_sems, recv_sems):
        my = lax.axis_index("i")
        left = lax.rem(my + (N_DEV - 1), N_DEV)
        right = lax.rem(my + 1, N_DEV)
        diag = lax.rem(my + 2, N_DEV)

        barrier_sem = pltpu.get_barrier_semaphore()
        for nbr in (left, right):
            pl.semaphore_signal(
                barrier_sem, inc=1,
                device_id=(nbr,), device_id_type=pl.DeviceIdType.MESH,
            )
        pl.semaphore_wait(barrier_sem, 2)

        comm_ref[my] = x_ref[...].astype(jnp.bfloat16)

        p1_r = pltpu.make_async_remote_copy(
            src_ref=comm_ref.at[my], dst_ref=comm_ref.at[my],
            send_sem=send_sems.at[0], recv_sem=recv_sems.at[0],
            device_id=(right,), device_id_type=pl.DeviceIdType.MESH,
        )
        p1_l = pltpu.make_async_remote_copy(
            src_ref=comm_ref.at[my], dst_ref=comm_ref.at[my],
            send_sem=send_sems.at[1], recv_sem=recv_sems.at[1],
            device_id=(left,), device_id_type=pl.DeviceIdType.MESH,
        )
        p1_r.start()
        p1_l.start()

        w_bf_ref[...] = w_ref[...].astype(jnp.bfloat16)
        out_ref[pl.ds(my * m_per, m_per), :] = jnp.dot(
            comm_ref[my], w_bf_ref[...], preferred_element_type=jnp.float32
        )

        p2_r = pltpu.make_async_remote_copy(
            src_ref=comm_ref.at[left, pl.ds(0, hm)],
            dst_ref=comm_ref.at[left, pl.ds(0, hm)],
            send_sem=send_sems.at[2], recv_sem=recv_sems.at[2],
            device_id=(right,), device_id_type=pl.DeviceIdType.MESH,
        )
        p2_l = pltpu.make_async_remote_copy(
            src_ref=comm_ref.at[right, pl.ds(hm, hm)],
            dst_ref=comm_ref.at[right, pl.ds(hm, hm)],
            send_sem=send_sems.at[3], recv_sem=recv_sems.at[3],
            device_id=(left,), device_id_type=pl.DeviceIdType.MESH,
        )

        p1_r.wait_recv()
        p2_r.start()
        p1_l.wait_recv()
        p2_l.start()

        out_ref[pl.ds(left * m_per, m_per), :] = jnp.dot(
            comm_ref[left], w_bf_ref[...], preferred_element_type=jnp.float32
        )
        out_ref[pl.ds(right * m_per, m_per), :] = jnp.dot(
            comm_ref[right], w_bf_ref[...], preferred_element_type=jnp.float32
        )

        p2_r.wait_recv()
        p2_l.wait_recv()
        out_ref[pl.ds(diag * m_per, m_per), :] = jnp.dot(
            comm_ref[diag], w_bf_ref[...], preferred_element_type=jnp.float32
        )

        p1_r.wait_send()
        p1_l.wait_send()
        p2_r.wait_send()
        p2_l.wait_send()

    return pl.pallas_call(
        body,
        out_shape=jax.ShapeDtypeStruct((N_DEV * m_per, n_per), jnp.float32),
        in_specs=[
            pl.BlockSpec(memory_space=pltpu.VMEM),
            pl.BlockSpec(memory_space=pltpu.VMEM),
        ],
        out_specs=pl.BlockSpec(memory_space=pltpu.VMEM),
        scratch_shapes=[
            pltpu.VMEM((N_DEV, m_per, k), jnp.bfloat16),
            pltpu.VMEM((k, n_per), jnp.bfloat16),
            pltpu.SemaphoreType.DMA((4,)),
            pltpu.SemaphoreType.DMA((4,)),
        ],
        compiler_params=pltpu.CompilerParams(collective_id=0),
    )(x, w_mat)


# device time: 46769 ns/iter; 1.8713x vs baseline; 1.0227x over previous
import jax
import jax.numpy as jnp
from jax import lax
from jax.experimental import pallas as pl
from jax.experimental.pallas import tpu as pltpu

N_DEV = 4


def kernel(x, w_mat):
    m_per, k = x.shape
    _, n_per = w_mat.shape
    hm = m_per // 2

    def body(x_ref, w_ref, out_ref, comm_ref, w_bf_ref, send_sems, recv_sems):
        my = lax.axis_index("i")
        left = lax.rem(my + (N_DEV - 1), N_DEV)
        right = lax.rem(my + 1, N_DEV)
        diag = lax.rem(my + 2, N_DEV)

        barrier_sem = pltpu.get_barrier_semaphore()
        for nbr in (left, right):
            pl.semaphore_signal(
                barrier_sem, inc=1,
                device_id=(nbr,), device_id_type=pl.DeviceIdType.MESH,
            )
        pl.semaphore_wait(barrier_sem, 2)

        comm_ref[my] = x_ref[...].astype(jnp.bfloat16)

        def rcopy(src_slot, row_lo, sem, dev):
            return pltpu.make_async_remote_copy(
                src_ref=comm_ref.at[src_slot, pl.ds(row_lo, hm)],
                dst_ref=comm_ref.at[src_slot, pl.ds(row_lo, hm)],
                send_sem=send_sems.at[sem], recv_sem=recv_sems.at[sem],
                device_id=(dev,), device_id_type=pl.DeviceIdType.MESH,
            )

        p1_r_top = rcopy(my, 0, 0, right)
        p1_l_bot = rcopy(my, hm, 2, left)
        p1_r_bot = rcopy(my, hm, 1, right)
        p1_l_top = rcopy(my, 0, 3, left)
        p1_r_top.start()
        p1_l_bot.start()
        p1_r_bot.start()
        p1_l_top.start()

        w_bf_ref[...] = w_ref[...].astype(jnp.bfloat16)
        out_ref[pl.ds(my * m_per, m_per), :] = jnp.dot(
            comm_ref[my], w_bf_ref[...], preferred_element_type=jnp.float32
        )

        p2_r = rcopy(left, 0, 4, right)
        p2_l = rcopy(right, hm, 5, left)

        p1_r_top.wait_recv()
        p2_r.start()
        p1_l_bot.wait_recv()
        p2_l.start()

        p1_r_bot.wait_recv()
        out_ref[pl.ds(left * m_per, m_per), :] = jnp.dot(
            comm_ref[left], w_bf_ref[...], preferred_element_type=jnp.float32
        )
        p1_l_top.wait_recv()
        out_ref[pl.ds(right * m_per, m_per), :] = jnp.dot(
            comm_ref[right], w_bf_ref[...], preferred_element_type=jnp.float32
        )

        p2_r.wait_recv()
        out_ref[pl.ds(diag * m_per, hm), :] = jnp.dot(
            comm_ref[diag, pl.ds(0, hm)], w_bf_ref[...],
            preferred_element_type=jnp.float32,
        )
        p2_l.wait_recv()
        out_ref[pl.ds(diag * m_per + hm, hm), :] = jnp.dot(
            comm_ref[diag, pl.ds(hm, hm)], w_bf_ref[...],
            preferred_element_type=jnp.float32,
        )

        for r in (p1_r_top, p1_l_bot, p1_r_bot, p1_l_top, p2_r, p2_l):
            r.wait_send()

    return pl.pallas_call(
        body,
        out_shape=jax.ShapeDtypeStruct((N_DEV * m_per, n_per), jnp.float32),
        in_specs=[
            pl.BlockSpec(memory_space=pltpu.VMEM),
            pl.BlockSpec(memory_space=pltpu.VMEM),
        ],
        out_specs=pl.BlockSpec(memory_space=pltpu.VMEM),
        scratch_shapes=[
            pltpu.VMEM((N_DEV, m_per, k), jnp.bfloat16),
            pltpu.VMEM((k, n_per), jnp.bfloat16),
            pltpu.SemaphoreType.DMA((6,)),
            pltpu.SemaphoreType.DMA((6,)),
        ],
        compiler_params=pltpu.CompilerParams(collective_id=0),
    )(x, w_mat)


# device time: 46533 ns/iter; 1.8808x vs baseline; 1.0051x over previous
import jax
import jax.numpy as jnp
from jax import lax
from jax.experimental import pallas as pl
from jax.experimental.pallas import tpu as pltpu

N_DEV = 4


def kernel(x, w_mat):
    m_per, k = x.shape
    _, n_per = w_mat.shape
    hm = m_per // 2

    def body(x_ref, w_ref, out_ref, comm_ref, w_bf_ref, send_sems, recv_sems):
        my = lax.axis_index("i")
        left = lax.rem(my + (N_DEV - 1), N_DEV)
        right = lax.rem(my + 1, N_DEV)
        diag = lax.rem(my + 2, N_DEV)

        barrier_sem = pltpu.get_barrier_semaphore()
        for nbr in (left, right):
            pl.semaphore_signal(
                barrier_sem, inc=1,
                device_id=(nbr,), device_id_type=pl.DeviceIdType.MESH,
            )
        comm_ref[my] = x_ref[...].astype(jnp.bfloat16)
        pl.semaphore_wait(barrier_sem, 2)

        def rcopy(src_slot, row_lo, sem, dev):
            return pltpu.make_async_remote_copy(
                src_ref=comm_ref.at[src_slot, pl.ds(row_lo, hm)],
                dst_ref=comm_ref.at[src_slot, pl.ds(row_lo, hm)],
                send_sem=send_sems.at[sem], recv_sem=recv_sems.at[sem],
                device_id=(dev,), device_id_type=pl.DeviceIdType.MESH,
            )

        p1_r_top = rcopy(my, 0, 0, right)
        p1_l_bot = rcopy(my, hm, 2, left)
        p1_r_bot = rcopy(my, hm, 1, right)
        p1_l_top = rcopy(my, 0, 3, left)
        p1_r_top.start()
        p1_l_bot.start()
        p1_r_bot.start()
        p1_l_top.start()

        w_bf_ref[...] = w_ref[...].astype(jnp.bfloat16)
        out_ref[pl.ds(my * m_per, m_per), :] = jnp.dot(
            comm_ref[my], w_bf_ref[...], preferred_element_type=jnp.float32
        )

        p2_r = rcopy(left, 0, 4, right)
        p2_l = rcopy(right, hm, 5, left)

        p1_r_top.wait_recv()
        p2_r.start()
        p1_l_bot.wait_recv()
        p2_l.start()

        p1_r_bot.wait_recv()
        out_ref[pl.ds(left * m_per, m_per), :] = jnp.dot(
            comm_ref[left], w_bf_ref[...], preferred_element_type=jnp.float32
        )
        p1_l_top.wait_recv()
        out_ref[pl.ds(right * m_per, m_per), :] = jnp.dot(
            comm_ref[right], w_bf_ref[...], preferred_element_type=jnp.float32
        )

        p2_r.wait_recv()
        out_ref[pl.ds(diag * m_per, hm), :] = jnp.dot(
            comm_ref[diag, pl.ds(0, hm)], w_bf_ref[...],
            preferred_element_type=jnp.float32,
        )
        p2_l.wait_recv()
        out_ref[pl.ds(diag * m_per + hm, hm), :] = jnp.dot(
            comm_ref[diag, pl.ds(hm, hm)], w_bf_ref[...],
            preferred_element_type=jnp.float32,
        )

        for r in (p1_r_top, p1_l_bot, p1_r_bot, p1_l_top, p2_r, p2_l):
            r.wait_send()

    return pl.pallas_call(
        body,
        out_shape=jax.ShapeDtypeStruct((N_DEV * m_per, n_per), jnp.float32),
        in_specs=[
            pl.BlockSpec(memory_space=pltpu.VMEM),
            pl.BlockSpec(memory_space=pltpu.VMEM),
        ],
        out_specs=pl.BlockSpec(memory_space=pltpu.VMEM),
        scratch_shapes=[
            pltpu.VMEM((N_DEV, m_per, k), jnp.bfloat16),
            pltpu.VMEM((k, n_per), jnp.bfloat16),
            pltpu.SemaphoreType.DMA((6,)),
            pltpu.SemaphoreType.DMA((6,)),
        ],
        compiler_params=pltpu.CompilerParams(collective_id=0),
    )(x, w_mat)


# device time: 46106 ns/iter; 1.8982x vs baseline; 1.0093x over previous
import jax
import jax.numpy as jnp
from jax import lax
from jax.experimental import pallas as pl
from jax.experimental.pallas import tpu as pltpu

N_DEV = 4


def kernel(x, w_mat):
    m_per, k = x.shape
    _, n_per = w_mat.shape
    hm = m_per // 2

    def body(x_ref, w_ref, out_ref, comm_ref, w_bf_ref, send_sems, recv_sems):
        my = lax.axis_index("i")
        left = lax.rem(my + (N_DEV - 1), N_DEV)
        right = lax.rem(my + 1, N_DEV)
        diag = lax.rem(my + 2, N_DEV)

        barrier_sem = pltpu.get_barrier_semaphore()
        for nbr in (left, right):
            pl.semaphore_signal(
                barrier_sem, inc=1,
                device_id=(nbr,), device_id_type=pl.DeviceIdType.MESH,
            )
        comm_ref[my] = x_ref[...].astype(jnp.bfloat16)
        pl.semaphore_wait(barrier_sem, 2)

        def rcopy(src_slot, row_lo, rows, sem, dev):
            return pltpu.make_async_remote_copy(
                src_ref=comm_ref.at[src_slot, pl.ds(row_lo, rows)],
                dst_ref=comm_ref.at[src_slot, pl.ds(row_lo, rows)],
                send_sem=send_sems.at[sem], recv_sem=recv_sems.at[sem],
                device_id=(dev,), device_id_type=pl.DeviceIdType.MESH,
            )

        qm = hm // 2

        p1_r_top = rcopy(my, 0, hm, 0, right)
        p1_l_bot = rcopy(my, hm, hm, 2, left)
        p1_r_bot = rcopy(my, hm, hm, 1, right)
        p1_l_top = rcopy(my, 0, hm, 3, left)
        p1_r_top.start()
        p1_l_bot.start()
        p1_r_bot.start()
        p1_l_top.start()

        w_bf_ref[...] = w_ref[...].astype(jnp.bfloat16)
        out_ref[pl.ds(my * m_per, m_per), :] = jnp.dot(
            comm_ref[my], w_bf_ref[...], preferred_element_type=jnp.float32
        )

        p2_r1 = rcopy(left, 0, qm, 4, right)
        p2_r2 = rcopy(left, qm, qm, 5, right)
        p2_l1 = rcopy(right, hm, qm, 6, left)
        p2_l2 = rcopy(right, hm + qm, qm, 7, left)

        p1_r_top.wait_recv()
        p2_r1.start()
        p2_r2.start()
        p1_l_bot.wait_recv()
        p2_l1.start()
        p2_l2.start()

        p1_r_bot.wait_recv()
        out_ref[pl.ds(left * m_per, m_per), :] = jnp.dot(
            comm_ref[left], w_bf_ref[...], preferred_element_type=jnp.float32
        )
        p1_l_top.wait_recv()
        out_ref[pl.ds(right * m_per, m_per), :] = jnp.dot(
            comm_ref[right], w_bf_ref[...], preferred_element_type=jnp.float32
        )

        for rdma, row_lo in (
            (p2_r1, 0), (p2_l1, hm), (p2_r2, qm), (p2_l2, hm + qm)
        ):
            rdma.wait_recv()
            out_ref[pl.ds(diag * m_per + row_lo, qm), :] = jnp.dot(
                comm_ref[diag, pl.ds(row_lo, qm)], w_bf_ref[...],
                preferred_element_type=jnp.float32,
            )

        for r in (p1_r_top, p1_l_bot, p1_r_bot, p1_l_top,
                  p2_r1, p2_r2, p2_l1, p2_l2):
            r.wait_send()

    return pl.pallas_call(
        body,
        out_shape=jax.ShapeDtypeStruct((N_DEV * m_per, n_per), jnp.float32),
        in_specs=[
            pl.BlockSpec(memory_space=pltpu.VMEM),
            pl.BlockSpec(memory_space=pltpu.VMEM),
        ],
        out_specs=pl.BlockSpec(memory_space=pltpu.VMEM),
        scratch_shapes=[
            pltpu.VMEM((N_DEV, m_per, k), jnp.bfloat16),
            pltpu.VMEM((k, n_per), jnp.bfloat16),
            pltpu.SemaphoreType.DMA((8,)),
            pltpu.SemaphoreType.DMA((8,)),
        ],
        compiler_params=pltpu.CompilerParams(collective_id=0),
    )(x, w_mat)
